# baseline (speedup 1.0000x reference)
import jax
import jax.numpy as jnp
import numpy as np
from jax.experimental import pallas as pl
from jax.experimental.pallas import tpu as pltpu

_TOPK = 512
_BM = 128

_INT_MIN = np.int32(-(2**31))


def _fused_kernel(xs_ref, wh_hbm, wl_hbm, b_ref, o_ref,
                  wh_s, wl_s, sem_h, sem_l):
    @pl.when(pl.program_id(0) == 0)
    def _load_w():
        cp_h = pltpu.make_async_copy(wh_hbm, wh_s, sem_h)
        cp_l = pltpu.make_async_copy(wl_hbm, wl_s, sem_l)
        cp_h.start()
        cp_l.start()
        cp_h.wait()
        cp_l.wait()

    dims = (((1,), (0,)), ((), ()))
    xs = xs_ref[...]
    d = jax.lax.dot_general(xs, wh_s[...], dims,
                            preferred_element_type=jnp.float32)
    acc = d[:_BM] + d[_BM:]
    acc = acc + jax.lax.dot_general(xs[:_BM], wl_s[...], dims,
                                    preferred_element_type=jnp.float32)
    y = acc + b_ref[...]
    y = jnp.where(y >= 0.0, y, 0.1 * y)

    for g in range(_BM // 8):
        yg = y[g * 8:(g + 1) * 8, :]
        ig = jax.lax.bitcast_convert_type(yg, jnp.int32)
        vg = jnp.where(ig >= 0, ig, ig ^ np.int32(0x7FFFFFFF))
        tb = jnp.zeros((8, 1), jnp.int32)
        for j in range(31, -1, -1):
            bit = np.uint32(1 << j).view(np.int32)
            cand = tb | bit
            cnt = jnp.sum((vg >= (cand ^ _INT_MIN)).astype(jnp.int32),
                          axis=1, keepdims=True)
            tb = jnp.where(cnt >= _TOPK, cand, tb)
        thr = tb ^ _INT_MIN
        o_ref[g * 8:(g + 1) * 8, :] = jnp.where(vg >= thr, yg, 0.0)


def kernel(input, W, b):
    m, k = input.shape
    n = W.shape[0]
    xh = input.astype(jnp.bfloat16)
    xl = (input - xh.astype(jnp.float32)).astype(jnp.bfloat16)
    nblk = m // _BM
    xs = jnp.concatenate(
        (xh.reshape(nblk, _BM, k), xl.reshape(nblk, _BM, k)),
        axis=1).reshape(nblk * 2 * _BM, k)
    wh = W.astype(jnp.bfloat16)
    wl = (W - wh.astype(jnp.float32)).astype(jnp.bfloat16)
    wht = wh.T
    wlt = wl.T
    b2 = b.reshape(1, n)

    grid = (nblk,)
    return pl.pallas_call(
        _fused_kernel,
        grid=grid,
        in_specs=[
            pl.BlockSpec((2 * _BM, k), lambda i: (i, 0)),
            pl.BlockSpec(memory_space=pl.ANY),
            pl.BlockSpec(memory_space=pl.ANY),
            pl.BlockSpec((1, n), lambda i: (0, 0)),
        ],
        out_specs=pl.BlockSpec((_BM, n), lambda i: (i, 0)),
        out_shape=jax.ShapeDtypeStruct((m, n), jnp.float32),
        scratch_shapes=[
            pltpu.VMEM((k, n), jnp.bfloat16),
            pltpu.VMEM((k, n), jnp.bfloat16),
            pltpu.SemaphoreType.DMA,
            pltpu.SemaphoreType.DMA,
        ],
        compiler_params=pltpu.CompilerParams(
            dimension_semantics=("parallel",),
        ),
    )(xs, wht, wlt, b2)


# speedup vs baseline: 1.0384x; 1.0384x over previous
import jax
import jax.numpy as jnp
import numpy as np
from jax.experimental import pallas as pl
from jax.experimental.pallas import tpu as pltpu

_TOPK = 512
_BM = 128

_INT_MIN = np.int32(-(2**31))


def _fused_kernel(xh_ref, xl_ref, wh_hbm, wl_hbm, b_ref, o_ref,
                  wh_s, wl_s, sem_h, sem_l):
    @pl.when(pl.program_id(0) == 0)
    def _load_w():
        cp_h = pltpu.make_async_copy(wh_hbm, wh_s, sem_h)
        cp_l = pltpu.make_async_copy(wl_hbm, wl_s, sem_l)
        cp_h.start()
        cp_l.start()
        cp_h.wait()
        cp_l.wait()

    dims = (((1,), (0,)), ((), ()))
    xh = xh_ref[...]
    xl = xl_ref[...]
    acc = jax.lax.dot_general(xh, wh_s[...], dims,
                              preferred_element_type=jnp.float32)
    acc = acc + jax.lax.dot_general(xh, wl_s[...], dims,
                                    preferred_element_type=jnp.float32)
    acc = acc + jax.lax.dot_general(xl, wh_s[...], dims,
                                    preferred_element_type=jnp.float32)
    y = acc + b_ref[...]
    y = jnp.where(y >= 0.0, y, 0.1 * y)

    i32 = jax.lax.bitcast_convert_type(y, jnp.int32)
    v = jnp.where(i32 >= 0, i32, i32 ^ np.int32(0x7FFFFFFF))

    tb = jnp.zeros((v.shape[0], 1), jnp.int32)
    for j in range(31, -1, -1):
        bit = np.uint32(1 << j).view(np.int32)
        cand = tb | bit
        cnt = jnp.sum((v >= (cand ^ _INT_MIN)).astype(jnp.int32),
                      axis=1, keepdims=True)
        tb = jnp.where(cnt >= _TOPK, cand, tb)
    thr = tb ^ _INT_MIN
    o_ref[...] = jnp.where(v >= thr, y, 0.0)


def kernel(input, W, b):
    m, k = input.shape
    n = W.shape[0]
    xh = input.astype(jnp.bfloat16)
    xl = (input - xh.astype(jnp.float32)).astype(jnp.bfloat16)
    wh = W.astype(jnp.bfloat16)
    wl = (W - wh.astype(jnp.float32)).astype(jnp.bfloat16)
    wht = wh.T
    wlt = wl.T
    b2 = b.reshape(1, n)

    grid = (m // _BM,)
    return pl.pallas_call(
        _fused_kernel,
        grid=grid,
        in_specs=[
            pl.BlockSpec((_BM, k), lambda i: (i, 0)),
            pl.BlockSpec((_BM, k), lambda i: (i, 0)),
            pl.BlockSpec(memory_space=pl.ANY),
            pl.BlockSpec(memory_space=pl.ANY),
            pl.BlockSpec((1, n), lambda i: (0, 0)),
        ],
        out_specs=pl.BlockSpec((_BM, n), lambda i: (i, 0)),
        out_shape=jax.ShapeDtypeStruct((m, n), jnp.float32),
        scratch_shapes=[
            pltpu.VMEM((k, n), jnp.bfloat16),
            pltpu.VMEM((k, n), jnp.bfloat16),
            pltpu.SemaphoreType.DMA,
            pltpu.SemaphoreType.DMA,
        ],
        compiler_params=pltpu.CompilerParams(
            dimension_semantics=("parallel",),
        ),
    )(xh, xl, wht, wlt, b2)
